# Initial kernel scaffold; baseline (speedup 1.0000x reference)
import jax
import jax.numpy as jnp
from jax.experimental import pallas as pl


def kernel(pred_logits, pred_boxes, gt_boxes_xyxy, image_size_xyxy, gt_labels):
    raise NotImplementedError("write your pallas kernel here")


# speedup vs baseline: 34.0606x; 34.0606x over previous
import functools

import jax
import jax.numpy as jnp
from jax import lax
from jax.experimental import pallas as pl

B, N, C, M = 2, 20000, 80, 100
MP = 128
BN = 1000
NBLK = N // BN
ALPHA, GAMMA = 0.25, 2.0
OTA_K = 10
FINF = float("inf")
IBIG = 2**30


def _cost_iou_block(lg, pb, gt, lab, img):
    prob = jax.nn.sigmoid(lg)
    ci = lax.broadcasted_iota(jnp.int32, (C, MP), 0)
    oh = (ci == lab).astype(jnp.float32)
    pos = lax.dot_general(prob, oh, (((1,), (0,)), ((), ())),
                          preferred_element_type=jnp.float32,
                          precision=lax.Precision.HIGHEST)
    neg = 1.0 - pos
    cost_class = (ALPHA * neg ** GAMMA * (-jnp.log(pos + 1e-8))
                  - (1.0 - ALPHA) * pos ** GAMMA * (-jnp.log(neg + 1e-8)))

    px1 = pb[:, 0:1]
    py1 = pb[:, 1:2]
    px2 = pb[:, 2:3]
    py2 = pb[:, 3:4]
    gx1 = gt[0:1, :]
    gy1 = gt[1:2, :]
    gx2 = gt[2:3, :]
    gy2 = gt[3:4, :]
    i0 = img[0, 0]
    i1 = img[0, 1]
    i2 = img[0, 2]
    i3 = img[0, 3]
    d0 = jnp.abs(px1 / i0 - gx1 / i0)
    d1 = jnp.abs(py1 / i1 - gy1 / i1)
    d2 = jnp.abs(px2 / i2 - gx2 / i2)
    d3 = jnp.abs(py2 / i3 - gy2 / i3)
    cost_bbox = (d0 + d2) + (d1 + d3)

    ltx = jnp.maximum(px1, gx1)
    lty = jnp.maximum(py1, gy1)
    rbx = jnp.minimum(px2, gx2)
    rby = jnp.minimum(py2, gy2)
    wi = jnp.maximum(rbx - ltx, 0.0)
    hi = jnp.maximum(rby - lty, 0.0)
    inter = wi * hi
    area_p = (px2 - px1) * (py2 - py1)
    area_g = (gx2 - gx1) * (gy2 - gy1)
    union = area_p + area_g - inter
    iou = inter / (union + 1e-8)
    lmx = jnp.minimum(px1, gx1)
    lmy = jnp.minimum(py1, gy1)
    rmx = jnp.maximum(px2, gx2)
    rmy = jnp.maximum(py2, gy2)
    we = jnp.maximum(rmx - lmx, 0.0)
    he = jnp.maximum(rmy - lmy, 0.0)
    area_e = we * he
    giou = iou - (area_e - union) / (area_e + 1e-8)
    cost = 1.0 * cost_class + 5.0 * cost_bbox + 2.0 * (1.0 - giou)
    return cost, iou


def _merge_top10_idx(x, base, tvv, tiv):
    io16 = lax.broadcasted_iota(jnp.int32, (16, MP), 0)
    iob = lax.broadcasted_iota(jnp.int32, (BN, MP), 0)
    ntv = jnp.full((16, MP), FINF, jnp.float32)
    nti = jnp.zeros((16, MP), jnp.int32)
    for j in range(OTA_K):
        m1 = jnp.min(tvv, axis=0, keepdims=True)
        m2 = jnp.min(x, axis=0, keepdims=True)
        v = jnp.minimum(m1, m2)
        from_tv = m1 <= m2
        r_tv = jnp.min(jnp.where(tvv == v, io16, IBIG), axis=0, keepdims=True)
        i_tv = jnp.min(jnp.where(io16 == r_tv, tiv, IBIG), axis=0, keepdims=True)
        r_x = jnp.min(jnp.where(x == v, iob, IBIG), axis=0, keepdims=True)
        i_x = r_x + base
        idx = jnp.where(from_tv, i_tv, i_x)
        ntv = jnp.where(io16 == j, v, ntv)
        nti = jnp.where(io16 == j, idx, nti)
        tvv = jnp.where(from_tv & (io16 == r_tv), FINF, tvv)
        x = jnp.where(jnp.logical_not(from_tv) & (iob == r_x), FINF, x)
    return ntv, nti


def _merge_top10_cnt(x, uvv, ucv):
    io16 = lax.broadcasted_iota(jnp.int32, (16, MP), 0)
    nuv = jnp.full((16, MP), FINF, jnp.float32)
    nuc = jnp.zeros((16, MP), jnp.int32)
    for j in range(OTA_K):
        m1 = jnp.min(uvv, axis=0, keepdims=True)
        m2 = jnp.min(x, axis=0, keepdims=True)
        v = jnp.minimum(m1, m2)
        cx = jnp.sum(jnp.where(x == v, 1, 0), axis=0, keepdims=True)
        cl = jnp.sum(jnp.where(uvv == v, ucv, 0), axis=0, keepdims=True)
        ntot = cx + cl
        nuv = jnp.where(io16 == j, v, nuv)
        nuc = jnp.where(io16 == j, ntot, nuc)
        uvv = jnp.where(uvv == v, FINF, uvv)
        x = jnp.where(x == v, FINF, x)
    return nuv, nuc


def _pass_a_kernel(lg_ref, pb_ref, gt_ref, lab_ref, img_ref,
                   cost_ref, sf_ref, si_ref,
                   tv, ti, uv, uc):
    i = pl.program_id(1)
    base = i * BN

    @pl.when(i == 0)
    def _init():
        tv[...] = jnp.full((16, MP), FINF, jnp.float32)
        ti[...] = jnp.zeros((16, MP), jnp.int32)
        uv[...] = jnp.full((16, MP), FINF, jnp.float32)
        uc[...] = jnp.zeros((16, MP), jnp.int32)

    cost, iou = _cost_iou_block(lg_ref[0], pb_ref[0], gt_ref[0], lab_ref[0], img_ref)
    cost_ref[0] = cost

    ntv, nti = _merge_top10_idx(cost, base, tv[...], ti[...])
    tv[...] = ntv
    ti[...] = nti
    nuv, nuc = _merge_top10_cnt(-iou, uv[...], uc[...])
    uv[...] = nuv
    uc[...] = nuc

    @pl.when(i == NBLK - 1)
    def _fin():
        io16 = lax.broadcasted_iota(jnp.int32, (16, MP), 0)
        uvv = uv[...]
        ucv = uc[...]
        taken = jnp.zeros((1, MP), jnp.int32)
        ssum = jnp.zeros((1, MP), jnp.float32)
        for j in range(OTA_K):
            vj = jnp.sum(jnp.where(io16 == j, uvv, 0.0), axis=0, keepdims=True)
            cj = jnp.sum(jnp.where(io16 == j, ucv, 0), axis=0, keepdims=True)
            tk = jnp.clip(jnp.minimum(cj, OTA_K - taken), 0, OTA_K)
            ssum = ssum + tk.astype(jnp.float32) * vj
            taken = taken + tk
        sum_iou = -ssum
        k = jnp.maximum(sum_iou.astype(jnp.int32), 1)
        tvv = tv[...]
        tiv = ti[...]
        vk = jnp.sum(jnp.where(io16 == (k - 1), tvv, 0.0), axis=0, keepdims=True)
        ik = jnp.sum(jnp.where(io16 == (k - 1), tiv, 0), axis=0, keepdims=True)
        v1 = jnp.sum(jnp.where(io16 == 0, tvv, 0.0), axis=0, keepdims=True)
        p0 = jnp.sum(jnp.where(io16 == 0, tiv, 0), axis=0, keepdims=True)
        zf = jnp.zeros((6, MP), jnp.float32)
        zi = jnp.zeros((6, MP), jnp.int32)
        sf_ref[0] = jnp.concatenate([vk, v1, zf], axis=0)
        si_ref[0] = jnp.concatenate([ik, p0, zi], axis=0)


def _mm_fixed(cost, base, vk, ik):
    rows = lax.broadcasted_iota(jnp.int32, (BN, MP), 0) + base
    lanes = lax.broadcasted_iota(jnp.int32, (BN, MP), 1)
    colm = lanes < M
    sel = (cost < vk) | ((cost == vk) & (rows <= ik))
    mm_f = jnp.where(sel & colm, 1.0, 0.0)
    rs = jnp.sum(mm_f, axis=1, keepdims=True)
    multi = rs > 1.0
    cmask = jnp.where(colm, cost, FINF)
    minr = jnp.min(cmask, axis=1, keepdims=True)
    mc = jnp.min(jnp.where(cmask == minr, lanes, IBIG), axis=1, keepdims=True)
    mmf = jnp.where(multi, jnp.where(lanes == mc, 1.0, 0.0), mm_f)
    return mmf, rows


def _pass_b_kernel(cost_ref, sf_ref, si_ref, aux_ref, acc):
    i = pl.program_id(1)
    base = i * BN

    @pl.when(i == 0)
    def _init():
        acc[...] = jnp.zeros((8, MP), jnp.float32)

    vk = sf_ref[0, 0:1, :]
    ik = si_ref[0, 0:1, :]
    p0 = si_ref[0, 1:2, :]
    mmf, rows = _mm_fixed(cost_ref[0], base, vk, ik)
    colsum = jnp.sum(mmf, axis=0, keepdims=True)
    selp = jnp.sum(jnp.where(rows == p0, mmf, 0.0), axis=0, keepdims=True)
    io8 = lax.broadcasted_iota(jnp.int32, (8, MP), 0)
    upd = jnp.where(io8 == 0, colsum, jnp.where(io8 == 1, selp, 0.0))
    acc[...] = acc[...] + upd

    @pl.when(i == NBLK - 1)
    def _fin():
        aux_ref[0] = acc[...]


def _pass_c_kernel(cost_ref, sf_ref, si_ref, aux_ref, out_ref):
    i = pl.program_id(1)
    base = i * BN
    vk = sf_ref[0, 0:1, :]
    ik = si_ref[0, 0:1, :]
    p0 = si_ref[0, 1:2, :]
    empty = aux_ref[0, 0:1, :] == 0.0
    cost = cost_ref[0]
    mmf, rows = _mm_fixed(cost, base, vk, ik)
    mm_final = jnp.where(empty, jnp.where(rows == p0, 1.0, 0.0), mmf)
    out = mm_final * cost
    out_ref[0] = out[:, :M]


@functools.partial(jax.jit, static_argnames=())
def kernel(pred_logits, pred_boxes, gt_boxes_xyxy, image_size_xyxy, gt_labels):
    f32 = jnp.float32
    dummy = jnp.array([0.0, 0.0, 8.0, 8.0], f32)
    gtb_pad = jnp.concatenate(
        [gt_boxes_xyxy.astype(f32), jnp.broadcast_to(dummy, (B, MP - M, 4))], axis=1)
    gtb_t = jnp.transpose(gtb_pad, (0, 2, 1))
    gtb_t = jnp.concatenate([gtb_t, jnp.zeros((B, 4, MP), f32)], axis=1)
    lab3 = jnp.concatenate(
        [gt_labels.astype(jnp.int32), jnp.zeros((B, MP - M), jnp.int32)], axis=1)[:, None, :]
    img = jnp.concatenate([image_size_xyxy.astype(f32), jnp.zeros((4,), f32)])[None, :]

    cost_buf, sf, si = _run_pass_a(pred_logits, pred_boxes, gtb_t, lab3, img)

    aux = _run_pass_b(cost_buf, sf, si)
    out = _run_pass_c(cost_buf, sf, si, aux)
    return out


def _run_pass_a(pred_logits, pred_boxes, gtb_t, lab3, img):
    from jax.experimental.pallas import tpu as pltpu
    return pl.pallas_call(
        _pass_a_kernel,
        grid=(B, NBLK),
        in_specs=[
            pl.BlockSpec((1, BN, C), lambda b, i: (b, i, 0)),
            pl.BlockSpec((1, BN, 4), lambda b, i: (b, i, 0)),
            pl.BlockSpec((1, 8, MP), lambda b, i: (b, 0, 0)),
            pl.BlockSpec((1, 1, MP), lambda b, i: (b, 0, 0)),
            pl.BlockSpec((1, 8), lambda b, i: (0, 0)),
        ],
        out_specs=[
            pl.BlockSpec((1, BN, MP), lambda b, i: (b, i, 0)),
            pl.BlockSpec((1, 8, MP), lambda b, i: (b, 0, 0)),
            pl.BlockSpec((1, 8, MP), lambda b, i: (b, 0, 0)),
        ],
        out_shape=[
            jax.ShapeDtypeStruct((B, N, MP), jnp.float32),
            jax.ShapeDtypeStruct((B, 8, MP), jnp.float32),
            jax.ShapeDtypeStruct((B, 8, MP), jnp.int32),
        ],
        scratch_shapes=[
            pltpu.VMEM((16, MP), jnp.float32),
            pltpu.VMEM((16, MP), jnp.int32),
            pltpu.VMEM((16, MP), jnp.float32),
            pltpu.VMEM((16, MP), jnp.int32),
        ],
    )(pred_logits, pred_boxes, gtb_t, lab3, img)


def _run_pass_b(cost_buf, sf, si):
    from jax.experimental.pallas import tpu as pltpu
    return pl.pallas_call(
        _pass_b_kernel,
        grid=(B, NBLK),
        in_specs=[
            pl.BlockSpec((1, BN, MP), lambda b, i: (b, i, 0)),
            pl.BlockSpec((1, 8, MP), lambda b, i: (b, 0, 0)),
            pl.BlockSpec((1, 8, MP), lambda b, i: (b, 0, 0)),
        ],
        out_specs=pl.BlockSpec((1, 8, MP), lambda b, i: (b, 0, 0)),
        out_shape=jax.ShapeDtypeStruct((B, 8, MP), jnp.float32),
        scratch_shapes=[pltpu.VMEM((8, MP), jnp.float32)],
    )(cost_buf, sf, si)


def _run_pass_c(cost_buf, sf, si, aux):
    return pl.pallas_call(
        _pass_c_kernel,
        grid=(B, NBLK),
        in_specs=[
            pl.BlockSpec((1, BN, MP), lambda b, i: (b, i, 0)),
            pl.BlockSpec((1, 8, MP), lambda b, i: (b, 0, 0)),
            pl.BlockSpec((1, 8, MP), lambda b, i: (b, 0, 0)),
            pl.BlockSpec((1, 8, MP), lambda b, i: (b, 0, 0)),
        ],
        out_specs=pl.BlockSpec((1, BN, M), lambda b, i: (b, i, 0)),
        out_shape=jax.ShapeDtypeStruct((B, N, M), jnp.float32),
    )(cost_buf, sf, si, aux)


# speedup vs baseline: 36.6803x; 1.0769x over previous
import functools

import jax
import jax.numpy as jnp
from jax import lax
from jax.experimental import pallas as pl

B, N, C, M = 2, 20000, 80, 100
MP = 128
BN = 5000
NBLK = N // BN
ALPHA, GAMMA = 0.25, 2.0
OTA_K = 10
FINF = float("inf")
IBIG = 2**30


def _cost_iou_block(lg, pb, gt, lab, img):
    prob = jax.nn.sigmoid(lg)
    ci = lax.broadcasted_iota(jnp.int32, (C, MP), 0)
    oh = (ci == lab).astype(jnp.float32)
    pos = lax.dot_general(prob, oh, (((1,), (0,)), ((), ())),
                          preferred_element_type=jnp.float32,
                          precision=lax.Precision.HIGHEST)
    neg = 1.0 - pos
    cost_class = (ALPHA * neg ** GAMMA * (-jnp.log(pos + 1e-8))
                  - (1.0 - ALPHA) * pos ** GAMMA * (-jnp.log(neg + 1e-8)))

    px1 = pb[:, 0:1]
    py1 = pb[:, 1:2]
    px2 = pb[:, 2:3]
    py2 = pb[:, 3:4]
    gx1 = gt[0:1, :]
    gy1 = gt[1:2, :]
    gx2 = gt[2:3, :]
    gy2 = gt[3:4, :]
    i0 = img[0, 0]
    i1 = img[0, 1]
    i2 = img[0, 2]
    i3 = img[0, 3]
    d0 = jnp.abs(px1 / i0 - gx1 / i0)
    d1 = jnp.abs(py1 / i1 - gy1 / i1)
    d2 = jnp.abs(px2 / i2 - gx2 / i2)
    d3 = jnp.abs(py2 / i3 - gy2 / i3)
    cost_bbox = (d0 + d2) + (d1 + d3)

    ltx = jnp.maximum(px1, gx1)
    lty = jnp.maximum(py1, gy1)
    rbx = jnp.minimum(px2, gx2)
    rby = jnp.minimum(py2, gy2)
    wi = jnp.maximum(rbx - ltx, 0.0)
    hi = jnp.maximum(rby - lty, 0.0)
    inter = wi * hi
    area_p = (px2 - px1) * (py2 - py1)
    area_g = (gx2 - gx1) * (gy2 - gy1)
    union = area_p + area_g - inter
    iou = inter / (union + 1e-8)
    lmx = jnp.minimum(px1, gx1)
    lmy = jnp.minimum(py1, gy1)
    rmx = jnp.maximum(px2, gx2)
    rmy = jnp.maximum(py2, gy2)
    we = jnp.maximum(rmx - lmx, 0.0)
    he = jnp.maximum(rmy - lmy, 0.0)
    area_e = we * he
    giou = iou - (area_e - union) / (area_e + 1e-8)
    cost = 1.0 * cost_class + 5.0 * cost_bbox + 2.0 * (1.0 - giou)
    return cost, iou


def _pass_a_kernel(lg_ref, pb_ref, gt_ref, lab_ref, img_ref,
                   cost_ref, sf_ref, si_ref,
                   tv, ti, uv, uc):
    i = pl.program_id(1)
    base = i * BN
    io16 = lax.broadcasted_iota(jnp.int32, (16, MP), 0)
    iob = lax.broadcasted_iota(jnp.int32, (BN, MP), 0)

    @pl.when(i == 0)
    def _init():
        tv[...] = jnp.full((16, MP), FINF, jnp.float32)
        ti[...] = jnp.zeros((16, MP), jnp.int32)
        uv[...] = jnp.full((16, MP), FINF, jnp.float32)
        uc[...] = jnp.zeros((16, MP), jnp.int32)

    cost, iou = _cost_iou_block(lg_ref[0], pb_ref[0], gt_ref[0], lab_ref[0], img_ref)
    cost_ref[0] = cost

    tvv = tv[...]
    tiv = ti[...]
    x = cost
    ntv = jnp.full((16, MP), FINF, jnp.float32)
    nti = jnp.zeros((16, MP), jnp.int32)
    for j in range(OTA_K):
        m1 = jnp.min(tvv, axis=0, keepdims=True)
        m2 = jnp.min(x, axis=0, keepdims=True)
        v = jnp.minimum(m1, m2)
        from_tv = m1 <= m2
        r_tv = jnp.min(jnp.where(tvv == v, io16, IBIG), axis=0, keepdims=True)
        i_tv = jnp.min(jnp.where(io16 == r_tv, tiv, IBIG), axis=0, keepdims=True)
        r_x = jnp.min(jnp.where(x == v, iob, IBIG), axis=0, keepdims=True)
        idx = jnp.where(from_tv, i_tv, r_x + base)
        ntv = jnp.where(io16 == j, v, ntv)
        nti = jnp.where(io16 == j, idx, nti)
        tvv = jnp.where(from_tv & (io16 == r_tv), FINF, tvv)
        x = jnp.where(jnp.logical_not(from_tv) & (iob == r_x), FINF, x)
    tv[...] = ntv
    ti[...] = nti

    uvv = uv[...]
    ucv = uc[...]
    x = -iou
    nuv = jnp.full((16, MP), FINF, jnp.float32)
    nuc = jnp.zeros((16, MP), jnp.int32)
    for j in range(OTA_K):
        m1 = jnp.min(uvv, axis=0, keepdims=True)
        m2 = jnp.min(x, axis=0, keepdims=True)
        v = jnp.minimum(m1, m2)
        cx = jnp.sum(jnp.where(x == v, 1, 0), axis=0, keepdims=True)
        cl = jnp.sum(jnp.where(uvv == v, ucv, 0), axis=0, keepdims=True)
        nuv = jnp.where(io16 == j, v, nuv)
        nuc = jnp.where(io16 == j, cx + cl, nuc)
        uvv = jnp.where(uvv == v, FINF, uvv)
        x = jnp.where(x == v, FINF, x)
    uv[...] = nuv
    uc[...] = nuc

    @pl.when(i == NBLK - 1)
    def _fin():
        io16 = lax.broadcasted_iota(jnp.int32, (16, MP), 0)
        uvv = uv[...]
        ucv = uc[...]
        taken = jnp.zeros((1, MP), jnp.int32)
        ssum = jnp.zeros((1, MP), jnp.float32)
        for j in range(OTA_K):
            vj = jnp.sum(jnp.where(io16 == j, uvv, 0.0), axis=0, keepdims=True)
            cj = jnp.sum(jnp.where(io16 == j, ucv, 0), axis=0, keepdims=True)
            tk = jnp.clip(jnp.minimum(cj, OTA_K - taken), 0, OTA_K)
            ssum = ssum + tk.astype(jnp.float32) * vj
            taken = taken + tk
        sum_iou = -ssum
        k = jnp.maximum(sum_iou.astype(jnp.int32), 1)
        tvv = tv[...]
        tiv = ti[...]
        vk = jnp.sum(jnp.where(io16 == (k - 1), tvv, 0.0), axis=0, keepdims=True)
        ik = jnp.sum(jnp.where(io16 == (k - 1), tiv, 0), axis=0, keepdims=True)
        v1 = jnp.sum(jnp.where(io16 == 0, tvv, 0.0), axis=0, keepdims=True)
        p0 = jnp.sum(jnp.where(io16 == 0, tiv, 0), axis=0, keepdims=True)
        zf = jnp.zeros((6, MP), jnp.float32)
        zi = jnp.zeros((6, MP), jnp.int32)
        sf_ref[0] = jnp.concatenate([vk, v1, zf], axis=0)
        si_ref[0] = jnp.concatenate([ik, p0, zi], axis=0)


def _mm_fixed(cost, base, vk, ik):
    rows = lax.broadcasted_iota(jnp.int32, (BN, MP), 0) + base
    lanes = lax.broadcasted_iota(jnp.int32, (BN, MP), 1)
    colm = lanes < M
    sel = (cost < vk) | ((cost == vk) & (rows <= ik))
    mm_f = jnp.where(sel & colm, 1.0, 0.0)
    rs = jnp.sum(mm_f, axis=1, keepdims=True)
    multi = rs > 1.0
    cmask = jnp.where(colm, cost, FINF)
    minr = jnp.min(cmask, axis=1, keepdims=True)
    mc = jnp.min(jnp.where(cmask == minr, lanes, IBIG), axis=1, keepdims=True)
    mmf = jnp.where(multi, jnp.where(lanes == mc, 1.0, 0.0), mm_f)
    mcn = jnp.where(multi, mc, -1)
    return mmf, rows, mcn


def _pass_b_kernel(cost_ref, sf_ref, si_ref, aux_ref, rf_ref, acc):
    i = pl.program_id(1)
    base = i * BN

    @pl.when(i == 0)
    def _init():
        acc[...] = jnp.zeros((8, MP), jnp.float32)

    vk = sf_ref[0, 0:1, :]
    ik = si_ref[0, 0:1, :]
    p0 = si_ref[0, 1:2, :]
    mmf, rows, mcn = _mm_fixed(cost_ref[0], base, vk, ik)
    rf_ref[0] = mcn
    colsum = jnp.sum(mmf, axis=0, keepdims=True)
    selp = jnp.sum(jnp.where(rows == p0, mmf, 0.0), axis=0, keepdims=True)
    io8 = lax.broadcasted_iota(jnp.int32, (8, MP), 0)
    upd = jnp.where(io8 == 0, colsum, jnp.where(io8 == 1, selp, 0.0))
    acc[...] = acc[...] + upd

    @pl.when(i == NBLK - 1)
    def _fin():
        aux_ref[0] = acc[...]


def _pass_c_kernel(cost_ref, sf_ref, si_ref, aux_ref, rf_ref, out_ref):
    i = pl.program_id(1)
    base = i * BN
    vk = sf_ref[0, 0:1, :]
    ik = si_ref[0, 0:1, :]
    p0 = si_ref[0, 1:2, :]
    empty = aux_ref[0, 0:1, :] == 0.0
    cost = cost_ref[0]
    rf = rf_ref[0]
    rows = lax.broadcasted_iota(jnp.int32, (BN, MP), 0) + base
    lanes = lax.broadcasted_iota(jnp.int32, (BN, MP), 1)
    colm = lanes < M
    sel = (cost < vk) | ((cost == vk) & (rows <= ik))
    mm_f = jnp.where(sel & colm, 1.0, 0.0)
    mmf = jnp.where(rf >= 0, jnp.where(lanes == rf, 1.0, 0.0), mm_f)
    mm_final = jnp.where(empty, jnp.where(rows == p0, 1.0, 0.0), mmf)
    out = mm_final * cost
    out_ref[0] = out[:, :M]


@functools.partial(jax.jit, static_argnames=())
def kernel(pred_logits, pred_boxes, gt_boxes_xyxy, image_size_xyxy, gt_labels):
    f32 = jnp.float32
    dummy = jnp.array([0.0, 0.0, 8.0, 8.0], f32)
    gtb_pad = jnp.concatenate(
        [gt_boxes_xyxy.astype(f32), jnp.broadcast_to(dummy, (B, MP - M, 4))], axis=1)
    gtb_t = jnp.transpose(gtb_pad, (0, 2, 1))
    gtb_t = jnp.concatenate([gtb_t, jnp.zeros((B, 4, MP), f32)], axis=1)
    lab3 = jnp.concatenate(
        [gt_labels.astype(jnp.int32), jnp.zeros((B, MP - M), jnp.int32)], axis=1)[:, None, :]
    img = jnp.concatenate([image_size_xyxy.astype(f32), jnp.zeros((4,), f32)])[None, :]

    cost_buf, sf, si = _run_pass_a(pred_logits, pred_boxes, gtb_t, lab3, img)

    aux, rowfix = _run_pass_b(cost_buf, sf, si)
    out = _run_pass_c(cost_buf, sf, si, aux, rowfix)
    return out


def _run_pass_a(pred_logits, pred_boxes, gtb_t, lab3, img):
    from jax.experimental.pallas import tpu as pltpu
    return pl.pallas_call(
        _pass_a_kernel,
        grid=(B, NBLK),
        in_specs=[
            pl.BlockSpec((1, BN, C), lambda b, i: (b, i, 0)),
            pl.BlockSpec((1, BN, 4), lambda b, i: (b, i, 0)),
            pl.BlockSpec((1, 8, MP), lambda b, i: (b, 0, 0)),
            pl.BlockSpec((1, 1, MP), lambda b, i: (b, 0, 0)),
            pl.BlockSpec((1, 8), lambda b, i: (0, 0)),
        ],
        out_specs=[
            pl.BlockSpec((1, BN, MP), lambda b, i: (b, i, 0)),
            pl.BlockSpec((1, 8, MP), lambda b, i: (b, 0, 0)),
            pl.BlockSpec((1, 8, MP), lambda b, i: (b, 0, 0)),
        ],
        out_shape=[
            jax.ShapeDtypeStruct((B, N, MP), jnp.float32),
            jax.ShapeDtypeStruct((B, 8, MP), jnp.float32),
            jax.ShapeDtypeStruct((B, 8, MP), jnp.int32),
        ],
        scratch_shapes=[
            pltpu.VMEM((16, MP), jnp.float32),
            pltpu.VMEM((16, MP), jnp.int32),
            pltpu.VMEM((16, MP), jnp.float32),
            pltpu.VMEM((16, MP), jnp.int32),
        ],
    )(pred_logits, pred_boxes, gtb_t, lab3, img)


def _run_pass_b(cost_buf, sf, si):
    from jax.experimental.pallas import tpu as pltpu
    return pl.pallas_call(
        _pass_b_kernel,
        grid=(B, NBLK),
        in_specs=[
            pl.BlockSpec((1, BN, MP), lambda b, i: (b, i, 0)),
            pl.BlockSpec((1, 8, MP), lambda b, i: (b, 0, 0)),
            pl.BlockSpec((1, 8, MP), lambda b, i: (b, 0, 0)),
        ],
        out_specs=[
            pl.BlockSpec((1, 8, MP), lambda b, i: (b, 0, 0)),
            pl.BlockSpec((1, BN, 1), lambda b, i: (b, i, 0)),
        ],
        out_shape=[
            jax.ShapeDtypeStruct((B, 8, MP), jnp.float32),
            jax.ShapeDtypeStruct((B, N, 1), jnp.int32),
        ],
        scratch_shapes=[pltpu.VMEM((8, MP), jnp.float32)],
    )(cost_buf, sf, si)


def _run_pass_c(cost_buf, sf, si, aux, rowfix):
    return pl.pallas_call(
        _pass_c_kernel,
        grid=(B, NBLK),
        in_specs=[
            pl.BlockSpec((1, BN, MP), lambda b, i: (b, i, 0)),
            pl.BlockSpec((1, 8, MP), lambda b, i: (b, 0, 0)),
            pl.BlockSpec((1, 8, MP), lambda b, i: (b, 0, 0)),
            pl.BlockSpec((1, 8, MP), lambda b, i: (b, 0, 0)),
            pl.BlockSpec((1, BN, 1), lambda b, i: (b, i, 0)),
        ],
        out_specs=pl.BlockSpec((1, BN, M), lambda b, i: (b, i, 0)),
        out_shape=jax.ShapeDtypeStruct((B, N, M), jnp.float32),
    )(cost_buf, sf, si, aux, rowfix)
